# baseline (speedup 1.0000x reference)
import jax
import jax.numpy as jnp
from jax.experimental import pallas as pl
from jax.experimental.pallas import tpu as pltpu

PROJ_WIDTH = 128
BASE = 10000.0
BS = 4096
H_BLK = 4


def _rope_body(sid_ref, perm_ref, x_ref, o_ref):
    sid = sid_ref[0, 0, :].astype(jnp.float32)
    d = jax.lax.broadcasted_iota(jnp.int32, (BS, PROJ_WIDTH), 1)
    even = (d % 2) == 0
    expnt = ((d // 2) * 2).astype(jnp.float32) * (1.0 / PROJ_WIDTH)
    theta = jnp.exp(-jnp.log(BASE) * expnt)
    ofs = jnp.where(even, 0.0, 0.5 * jnp.pi)
    m = jnp.cos(sid[:, None] * theta - ofs)
    c = jnp.where(even, m, pltpu.roll(m, 1, 1))
    s = jnp.where(even, -pltpu.roll(m, PROJ_WIDTH - 1, 1), m)
    p = perm_ref[...]
    for i in range(H_BLK):
        xi = x_ref[0, i]
        swp = jnp.dot(xi, p, preferred_element_type=jnp.float32)
        o_ref[0, i] = c * xi + s * swp


@jax.jit
def kernel(x, seq_id):
    B, H1, H2, S, W = x.shape
    H = H1 * H2
    n_sblk = S // BS
    xr = x.reshape(B, H, S, W)
    sid = seq_id.reshape(B * n_sblk, 1, BS)
    j = jnp.arange(W)
    perm = (j[:, None] ^ 1 == j[None, :]).astype(jnp.float32)

    out = pl.pallas_call(
        _rope_body,
        grid=(B, n_sblk, H // H_BLK),
        in_specs=[
            pl.BlockSpec((1, 1, BS), lambda b, sblk, h: (b * n_sblk + sblk, 0, 0)),
            pl.BlockSpec((W, W), lambda b, sblk, h: (0, 0)),
            pl.BlockSpec((1, H_BLK, BS, W), lambda b, sblk, h: (b, h, sblk, 0)),
        ],
        out_specs=pl.BlockSpec((1, H_BLK, BS, W), lambda b, sblk, h: (b, h, sblk, 0)),
        out_shape=jax.ShapeDtypeStruct((B, H, S, W), jnp.float32),
        compiler_params=pltpu.CompilerParams(
            vmem_limit_bytes=63 * 1024 * 1024,
        ),
    )(sid, perm, xr)
    return out.reshape(B, H1, H2, S, W)


# speedup vs baseline: 1.5182x; 1.5182x over previous
import jax
import jax.numpy as jnp
from jax.experimental import pallas as pl
from jax.experimental.pallas import tpu as pltpu

PROJ_WIDTH = 128
BASE = 10000.0
SEQ = 4096
BS = 4096
H_BLK = 4


def _rope_body(sid_ref, perm_ref, x_ref, o_ref, cos_ref, sin_ref):
    h = pl.program_id(2)

    @pl.when(h == 0)
    def _compute_trig():
        sid = sid_ref[0, 0, :].astype(jnp.float32)
        d = jax.lax.broadcasted_iota(jnp.int32, (BS, PROJ_WIDTH), 1)
        expnt = ((d // 2) * 2).astype(jnp.float32) * (1.0 / PROJ_WIDTH)
        theta = jnp.exp(-jnp.log(BASE) * expnt)
        angle = sid[:, None] * theta
        cos_ref[...] = jnp.cos(angle)
        sg = jnp.where((d % 2) == 0, -1.0, 1.0)
        sin_ref[...] = sg * jnp.sin(angle)

    c = cos_ref[...]
    s = sin_ref[...]
    p = perm_ref[...]
    for i in range(H_BLK):
        xi = x_ref[0, i]
        swp = jnp.dot(xi, p, preferred_element_type=jnp.float32)
        o_ref[0, i] = c * xi + s * swp


@jax.jit
def kernel(x, seq_id):
    B, H1, H2, S, W = x.shape
    H = H1 * H2
    n_sblk = S // BS
    xr = x.reshape(B, H, S, W)
    sid = seq_id.reshape(B * n_sblk, 1, BS)
    j = jnp.arange(W)
    perm = (j[:, None] ^ 1 == j[None, :]).astype(jnp.float32)

    out = pl.pallas_call(
        _rope_body,
        grid=(B, n_sblk, H // H_BLK),
        in_specs=[
            pl.BlockSpec((1, 1, BS), lambda b, sblk, h: (b * n_sblk + sblk, 0, 0)),
            pl.BlockSpec((W, W), lambda b, sblk, h: (0, 0)),
            pl.BlockSpec((1, H_BLK, BS, W), lambda b, sblk, h: (b, h, sblk, 0)),
        ],
        out_specs=pl.BlockSpec((1, H_BLK, BS, W), lambda b, sblk, h: (b, h, sblk, 0)),
        out_shape=jax.ShapeDtypeStruct((B, H, S, W), jnp.float32),
        scratch_shapes=[
            pltpu.VMEM((BS, W), jnp.float32),
            pltpu.VMEM((BS, W), jnp.float32),
        ],
        compiler_params=pltpu.CompilerParams(
            vmem_limit_bytes=63 * 1024 * 1024,
        ),
    )(sid, perm, xr)
    return out.reshape(B, H1, H2, S, W)
